# baseline (speedup 1.0000x reference)
import functools

import jax
import jax.numpy as jnp
from jax import lax
from jax.experimental import pallas as pl
from jax.experimental.pallas import tpu as pltpu
from jax.experimental.pallas import tpu_sc as plsc

DIM = 32
B = 16384
L = 50

NC = 2
NS = 16
NW = NC * NS
E_W = B // NW
CHUNK_E = 32
N_CHUNK = E_W // CHUNK_E
ROWS_C = CHUNK_E * L


VOCAB = 1000000
CW = 512
K_W = 61
EXTRA_CID = K_W * NW
TAIL_SRC0 = VOCAB - 128
SKEW = 34


def _format_body(tabt_hbm, tail_hbm, flat_hbm, src_v, tail_v, dst_v, tmp_v,
                 lsem, osem):
    wid = lax.axis_index("s") * NC + lax.axis_index("c")
    d_lo = lax.iota(jnp.int32, 16)
    d_hi = d_lo + 16

    def col0_of(k):
        return pl.multiple_of((wid + k * NW) * CW, CW)

    def src_slice(slot):
        return src_v.at[pl.ds(pl.multiple_of(slot * DIM, DIM), DIM), :]

    def dst_slice(slot):
        return dst_v.at[pl.ds(pl.multiple_of(slot * (CW * DIM), 8), CW * DIM)]

    def start_load(k, slot):
        pltpu.async_copy(tabt_hbm.at[:, pl.ds(col0_of(k), CW)],
                         src_slice(slot), lsem.at[slot])

    def wait_load(k, slot):
        pltpu.make_async_copy(tabt_hbm.at[:, pl.ds(col0_of(k), CW)],
                              src_slice(slot), lsem.at[slot]).wait()

    lane = lax.iota(jnp.int32, 16)
    sc_base = lane * SKEW

    def transpose_rows(src, d_base, dst_base, n16):
        G = 8

        def blk(r16, carry):
            c0 = r16 * 16
            prev = None
            for g0 in range(0, DIM + G, G):
                cur = ([src[d_base + d, pl.ds(c0, 16)]
                        for d in range(g0, g0 + G)] if g0 < DIM else None)
                if prev is not None:
                    for j, d in enumerate(range(g0 - G, g0)):
                        plsc.store_scatter(tmp_v, [sc_base + d], prev[j])
                prev = cur
            prevw = None
            for t0 in range(0, 20, 4):
                curw = None
                if t0 < 16:
                    curw = []
                    for t in range(t0, t0 + 4):
                        i0 = t * SKEW
                        curw.append((t, plsc.load_gather(tmp_v, [lane + i0]),
                                     plsc.load_gather(tmp_v,
                                                      [lane + (i0 + 16)])))
                if prevw is not None:
                    for t, w0, w1 in prevw:
                        dst_v[pl.ds(dst_base + (c0 + t) * DIM, 16)] = w0
                        dst_v[pl.ds(dst_base + (c0 + t) * DIM + 16, 16)] = w1
                prevw = curw
            return carry

        lax.fori_loop(0, n16, blk, 0)

    def start_out(k, slot):
        pltpu.async_copy(dst_slice(slot),
                         flat_hbm.at[pl.ds(col0_of(k) * DIM, CW * DIM)],
                         osem.at[slot])

    def wait_out(k, slot):
        pltpu.make_async_copy(dst_slice(slot),
                              flat_hbm.at[pl.ds(col0_of(k) * DIM, CW * DIM)],
                              osem.at[slot]).wait()

    start_load(0, 0)
    start_load(1, 1)

    def body(k, carry):
        slot = k % 3
        wait_load(k, slot)

        @pl.when(k + 2 < K_W)
        def _():
            start_load(k + 2, (k + 2) % 3)

        @pl.when(k >= 3)
        def _():
            wait_out(k - 3, slot)

        transpose_rows(src_v, slot * DIM, slot * (CW * DIM), CW // 16)
        start_out(k, slot)
        return carry

    lax.fori_loop(0, K_W, body, 0)
    wait_out(K_W - 3, (K_W - 3) % 3)
    wait_out(K_W - 2, (K_W - 2) % 3)
    wait_out(K_W - 1, (K_W - 1) % 3)

    @pl.when(wid == 16)
    def _():
        c0 = EXTRA_CID * CW
        pltpu.sync_copy(tabt_hbm.at[:, pl.ds(c0, CW)], src_slice(0))
        transpose_rows(src_v, 0, 0, CW // 16)
        pltpu.sync_copy(dst_slice(0),
                        flat_hbm.at[pl.ds(c0 * DIM, CW * DIM)])
        pltpu.sync_copy(tail_hbm, tail_v)
        transpose_rows(tail_v, 0, 0, 128 // 16)
        pltpu.sync_copy(dst_v.at[pl.ds(0, 128 * DIM)],
                        flat_hbm.at[pl.ds(TAIL_SRC0 * DIM, 128 * DIM)])


_format = functools.partial(
    pl.kernel,
    mesh=plsc.VectorSubcoreMesh(core_axis_name="c", subcore_axis_name="s"),
    out_type=jax.ShapeDtypeStruct((VOCAB * DIM,), jnp.float32),
    scratch_types=[
        pltpu.VMEM((3 * DIM, CW), jnp.float32),
        pltpu.VMEM((DIM, 128), jnp.float32),
        pltpu.VMEM((3 * CW * DIM,), jnp.float32),
        pltpu.VMEM((16 * SKEW,), jnp.float32),
        pltpu.SemaphoreType.DMA((3,)),
        pltpu.SemaphoreType.DMA((3,)),
    ],
    compiler_params=pltpu.CompilerParams(use_tc_tiling_on_sc=True,
                                         needs_layout_passes=False),
)(_format_body)


def _pool_body(x_hbm, table_hbm, out_hbm, idx_v, rows_v, pooled_v, gsem):
    wid = lax.axis_index("s") * NC + lax.axis_index("c")
    ebase = wid * E_W

    def stage(c, slot):
        e0 = ebase + c * CHUNK_E
        ib = pl.multiple_of(slot * CHUNK_E, CHUNK_E)
        pltpu.sync_copy(x_hbm.at[pl.ds(e0, CHUNK_E), :],
                        idx_v.at[pl.ds(ib, CHUNK_E), :])
        for j in range(CHUNK_E):
            pltpu.async_copy(
                table_hbm.at[idx_v.at[ib + j]],
                rows_v.at[pl.ds(pl.multiple_of(slot * ROWS_C, 8) + j * L, L)],
                gsem.at[slot])

    def drain(slot):
        pltpu.make_async_copy(
            table_hbm.at[pl.ds(0, ROWS_C)],
            rows_v.at[pl.ds(pl.multiple_of(slot * ROWS_C, 8), ROWS_C)],
            gsem.at[slot]).wait()

    stage(0, 0)

    def chunk_body(c, carry):
        slot = c % 2
        e0 = ebase + c * CHUNK_E

        @pl.when(c + 1 < N_CHUNK)
        def _():
            stage(c + 1, (c + 1) % 2)

        drain(slot)
        rbase = slot * ROWS_C

        def elem_body(e, carry2):
            base = rbase + e * L
            acc0 = rows_v[base, pl.ds(0, 16)]
            acc1 = rows_v[base, pl.ds(16, 16)]
            for r in range(1, L):
                acc0 = acc0 + rows_v[base + r, pl.ds(0, 16)]
                acc1 = acc1 + rows_v[base + r, pl.ds(16, 16)]
            pooled_v[e, pl.ds(0, 16)] = acc0 * (1.0 / L)
            pooled_v[e, pl.ds(16, 16)] = acc1 * (1.0 / L)
            return carry2

        lax.fori_loop(0, CHUNK_E, elem_body, 0)
        pltpu.sync_copy(pooled_v, out_hbm.at[pl.ds(e0, CHUNK_E)])
        return carry

    lax.fori_loop(0, N_CHUNK, chunk_body, 0)


_pool = functools.partial(
    pl.kernel,
    mesh=plsc.VectorSubcoreMesh(core_axis_name="c", subcore_axis_name="s"),
    out_type=jax.ShapeDtypeStruct((B, DIM), jnp.float32),
    scratch_types=[
        pltpu.VMEM((2 * CHUNK_E, L), jnp.int32),
        pltpu.VMEM((2 * ROWS_C, DIM), jnp.float32),
        pltpu.VMEM((CHUNK_E, DIM), jnp.float32),
        pltpu.SemaphoreType.DMA((2,)),
    ],
    compiler_params=pltpu.CompilerParams(use_tc_tiling_on_sc=False),
)(_pool_body)


def _tail_body(pooled_ref, w_ref, b_ref, gamma_ref, beta_ref, mean_ref,
               var_ref, out_ref):
    p = pooled_ref[...]
    h = jnp.dot(p, w_ref[...], preferred_element_type=jnp.float32) + b_ref[...]
    scale = gamma_ref[...] * lax.rsqrt(var_ref[...] + 1e-3)
    h = (h - mean_ref[...]) * scale + beta_ref[...]
    nrm = lax.rsqrt(jnp.maximum(jnp.sum(h * h, axis=1, keepdims=True), 1e-12))
    out_ref[...] = h * nrm


def _tail(pooled, w, b, gamma, beta, mean, var):
    blk = 2048
    vec = pl.BlockSpec((1, DIM), lambda i: (0, 0))
    return pl.pallas_call(
        _tail_body,
        grid=(B // blk,),
        in_specs=[
            pl.BlockSpec((blk, DIM), lambda i: (i, 0)),
            pl.BlockSpec((DIM, DIM), lambda i: (0, 0)),
            vec, vec, vec, vec, vec,
        ],
        out_specs=pl.BlockSpec((blk, DIM), lambda i: (i, 0)),
        out_shape=jax.ShapeDtypeStruct((B, DIM), jnp.float32),
    )(pooled, w, b, gamma, beta, mean, var)


def kernel(x, table, W, b, gamma, beta, moving_mean, moving_var):
    tabt = jnp.swapaxes(table, 0, 1)
    flat = _format(tabt, lax.slice(tabt, (0, TAIL_SRC0), (DIM, VOCAB)))
    pooled = _pool(x.astype(jnp.int32), flat.reshape(VOCAB, DIM))
    r = lambda v: v.reshape(1, DIM)
    return _tail(pooled, W, r(b), r(gamma), r(beta), r(moving_mean),
                 r(moving_var))


# speedup vs baseline: 1.1204x; 1.1204x over previous
import functools

import jax
import jax.numpy as jnp
from jax import lax
from jax.experimental import pallas as pl
from jax.experimental.pallas import tpu as pltpu
from jax.experimental.pallas import tpu_sc as plsc

DIM = 32
B = 16384
L = 50

NC = 2
NS = 16
NW = NC * NS
E_W = B // NW
CHUNK_E = 32
N_CHUNK = E_W // CHUNK_E
ROWS_C = CHUNK_E * L


VOCAB = 1000000
CW = 512
K_W = 61
EXTRA_CID = K_W * NW
TAIL_SRC0 = VOCAB - 128
SKEW = 34
HDIM = DIM // 2


def _format_body(tabt_hbm, tail_hbm, flat_hbm, src_v, tail_v, dst_v, tmp_v,
                 lsem, osem):
    wid = lax.axis_index("s") * NC + lax.axis_index("c")
    d_lo = lax.iota(jnp.int32, 16)
    d_hi = d_lo + 16

    def col0_of(k):
        return pl.multiple_of((wid + k * NW) * CW, CW)

    def src_slice(slot):
        return src_v.at[pl.ds(pl.multiple_of(slot * DIM, DIM), DIM), :]

    def dst_slice(slot):
        return dst_v.at[pl.ds(pl.multiple_of(slot * (CW * HDIM), 8),
                              CW * HDIM)]

    def start_load(k, slot):
        pltpu.async_copy(tabt_hbm.at[:, pl.ds(col0_of(k), CW)],
                         src_slice(slot), lsem.at[slot])

    def wait_load(k, slot):
        pltpu.make_async_copy(tabt_hbm.at[:, pl.ds(col0_of(k), CW)],
                              src_slice(slot), lsem.at[slot]).wait()

    lane = lax.iota(jnp.int32, 16)
    sc_base = lane * SKEW

    def transpose_rows(src, d_base, dst_base, n16):
        G = 8

        def blk(r16, carry):
            c0 = r16 * 16
            prev = None
            for g0 in range(0, DIM + G, G):
                cur = ([src[d_base + d, pl.ds(c0, 16)]
                        for d in range(g0, g0 + G)] if g0 < DIM else None)
                if prev is not None:
                    for j, d in enumerate(range(g0 - G, g0)):
                        plsc.store_scatter(tmp_v, [sc_base + d], prev[j])
                prev = cur
            prevw = None
            for t0 in range(0, 20, 4):
                curw = None
                if t0 < 16:
                    curw = []
                    for t in range(t0, t0 + 4):
                        i0 = t * SKEW
                        curw.append((t, plsc.load_gather(tmp_v, [lane + i0]),
                                     plsc.load_gather(tmp_v,
                                                      [lane + (i0 + 16)])))
                if prevw is not None:
                    for t, w0, w1 in prevw:
                        pk = plsc.pack(w0, w1,
                                       format=plsc.PackFormat.INTERLEAVED)
                        pki = plsc.bitcast(pk, jnp.int32)
                        dst_v[pl.ds(dst_base + (c0 + t) * HDIM, 16)] = pki
                prevw = curw
            return carry

        lax.fori_loop(0, n16, blk, 0)

    def start_out(k, slot):
        pltpu.async_copy(dst_slice(slot),
                         flat_hbm.at[pl.ds(col0_of(k) * HDIM, CW * HDIM)],
                         osem.at[slot])

    def wait_out(k, slot):
        pltpu.make_async_copy(dst_slice(slot),
                              flat_hbm.at[pl.ds(col0_of(k) * HDIM, CW * HDIM)],
                              osem.at[slot]).wait()

    start_load(0, 0)

    def body(k, carry):
        slot = k % 2
        wait_load(k, slot)

        @pl.when(k + 1 < K_W)
        def _():
            start_load(k + 1, (k + 1) % 2)

        @pl.when(k >= 2)
        def _():
            wait_out(k - 2, slot)

        transpose_rows(src_v, slot * DIM, slot * (CW * HDIM), CW // 16)
        start_out(k, slot)
        return carry

    lax.fori_loop(0, K_W, body, 0)
    wait_out(K_W - 2, (K_W - 2) % 2)
    wait_out(K_W - 1, (K_W - 1) % 2)

    @pl.when(wid == 16)
    def _():
        c0 = EXTRA_CID * CW
        pltpu.sync_copy(tabt_hbm.at[:, pl.ds(c0, CW)], src_slice(0))
        transpose_rows(src_v, 0, 0, CW // 16)
        pltpu.sync_copy(dst_slice(0),
                        flat_hbm.at[pl.ds(c0 * HDIM, CW * HDIM)])
        pltpu.sync_copy(tail_hbm, tail_v)
        transpose_rows(tail_v, 0, 0, 128 // 16)
        pltpu.sync_copy(dst_v.at[pl.ds(0, 128 * HDIM)],
                        flat_hbm.at[pl.ds(TAIL_SRC0 * HDIM, 128 * HDIM)])


_format = functools.partial(
    pl.kernel,
    mesh=plsc.VectorSubcoreMesh(core_axis_name="c", subcore_axis_name="s"),
    out_type=jax.ShapeDtypeStruct((VOCAB * HDIM,), jnp.int32),
    scratch_types=[
        pltpu.VMEM((2 * DIM, CW), jnp.float32),
        pltpu.VMEM((DIM, 128), jnp.float32),
        pltpu.VMEM((2 * CW * HDIM,), jnp.int32),
        pltpu.VMEM((16 * SKEW,), jnp.float32),
        pltpu.SemaphoreType.DMA((2,)),
        pltpu.SemaphoreType.DMA((2,)),
    ],
    compiler_params=pltpu.CompilerParams(use_tc_tiling_on_sc=True,
                                         needs_layout_passes=False),
)(_format_body)


def _pool_body(x_hbm, table_hbm, out_hbm, idx_v, rows_v, pooled_v, gsem):
    wid = lax.axis_index("s") * NC + lax.axis_index("c")
    ebase = wid * E_W

    def stage(c, slot):
        e0 = ebase + c * CHUNK_E
        ib = pl.multiple_of(slot * CHUNK_E, CHUNK_E)
        pltpu.sync_copy(x_hbm.at[pl.ds(e0, CHUNK_E), :],
                        idx_v.at[pl.ds(ib, CHUNK_E), :])
        for j in range(CHUNK_E):
            pltpu.async_copy(
                table_hbm.at[idx_v.at[ib + j]],
                rows_v.at[pl.ds(pl.multiple_of(slot * ROWS_C, 8) + j * L, L)],
                gsem.at[slot])

    def drain(slot):
        pltpu.make_async_copy(
            table_hbm.at[pl.ds(0, ROWS_C)],
            rows_v.at[pl.ds(pl.multiple_of(slot * ROWS_C, 8), ROWS_C)],
            gsem.at[slot]).wait()

    stage(0, 0)

    def chunk_body(c, carry):
        slot = c % 2
        e0 = ebase + c * CHUNK_E

        @pl.when(c + 1 < N_CHUNK)
        def _():
            stage(c + 1, (c + 1) % 2)

        drain(slot)
        rbase = slot * ROWS_C

        def elem_body(e, carry2):
            base = rbase + e * L
            acc0, acc1 = plsc.unpack(
                plsc.bitcast(rows_v[base, :], jnp.bfloat16),
                format=plsc.PackFormat.INTERLEAVED)
            for r in range(1, L):
                a, b = plsc.unpack(
                    plsc.bitcast(rows_v[base + r, :], jnp.bfloat16),
                    format=plsc.PackFormat.INTERLEAVED)
                acc0 = acc0 + a
                acc1 = acc1 + b
            pooled_v[e, pl.ds(0, 16)] = acc0 * (1.0 / L)
            pooled_v[e, pl.ds(16, 16)] = acc1 * (1.0 / L)
            return carry2

        lax.fori_loop(0, CHUNK_E, elem_body, 0)
        pltpu.sync_copy(pooled_v, out_hbm.at[pl.ds(e0, CHUNK_E)])
        return carry

    lax.fori_loop(0, N_CHUNK, chunk_body, 0)


_pool = functools.partial(
    pl.kernel,
    mesh=plsc.VectorSubcoreMesh(core_axis_name="c", subcore_axis_name="s"),
    out_type=jax.ShapeDtypeStruct((B, DIM), jnp.float32),
    scratch_types=[
        pltpu.VMEM((2 * CHUNK_E, L), jnp.int32),
        pltpu.VMEM((2 * ROWS_C, HDIM), jnp.int32),
        pltpu.VMEM((CHUNK_E, DIM), jnp.float32),
        pltpu.SemaphoreType.DMA((2,)),
    ],
    compiler_params=pltpu.CompilerParams(use_tc_tiling_on_sc=False,
                                         needs_layout_passes=False),
)(_pool_body)


def _tail_body(pooled_ref, w_ref, b_ref, gamma_ref, beta_ref, mean_ref,
               var_ref, out_ref):
    p = pooled_ref[...]
    h = jnp.dot(p, w_ref[...], preferred_element_type=jnp.float32) + b_ref[...]
    scale = gamma_ref[...] * lax.rsqrt(var_ref[...] + 1e-3)
    h = (h - mean_ref[...]) * scale + beta_ref[...]
    nrm = lax.rsqrt(jnp.maximum(jnp.sum(h * h, axis=1, keepdims=True), 1e-12))
    out_ref[...] = h * nrm


def _tail(pooled, w, b, gamma, beta, mean, var):
    blk = 2048
    vec = pl.BlockSpec((1, DIM), lambda i: (0, 0))
    return pl.pallas_call(
        _tail_body,
        grid=(B // blk,),
        in_specs=[
            pl.BlockSpec((blk, DIM), lambda i: (i, 0)),
            pl.BlockSpec((DIM, DIM), lambda i: (0, 0)),
            vec, vec, vec, vec, vec,
        ],
        out_specs=pl.BlockSpec((blk, DIM), lambda i: (i, 0)),
        out_shape=jax.ShapeDtypeStruct((B, DIM), jnp.float32),
    )(pooled, w, b, gamma, beta, mean, var)


def kernel(x, table, W, b, gamma, beta, moving_mean, moving_var):
    tabt = jnp.swapaxes(table, 0, 1)
    flat = _format(tabt, lax.slice(tabt, (0, TAIL_SRC0), (DIM, VOCAB)))
    pooled = _pool(x.astype(jnp.int32), flat.reshape(VOCAB, HDIM))
    r = lambda v: v.reshape(1, DIM)
    return _tail(pooled, W, r(b), r(gamma), r(beta), r(moving_mean),
                 r(moving_var))


# speedup vs baseline: 1.5191x; 1.3559x over previous
import functools

import jax
import jax.numpy as jnp
from jax import lax
from jax.experimental import pallas as pl
from jax.experimental.pallas import tpu as pltpu
from jax.experimental.pallas import tpu_sc as plsc

DIM = 32
B = 16384
L = 50

NC = 2
NS = 16
NW = NC * NS
E_W = B // NW
CHUNK_E = 32
N_CHUNK = E_W // CHUNK_E
ROWS_C = CHUNK_E * L


VOCAB = 1000000
CW = 512
K_W = 61
EXTRA_CID = K_W * NW
TAIL_SRC0 = VOCAB - 128
SKEW = 34
HDIM = DIM // 2


def _format_body(tabt_hbm, tail_hbm, flat_hbm, src_v, tail_v, dst_v, tmp_v,
                 lsem, osem):
    wid = lax.axis_index("s") * NC + lax.axis_index("c")
    d_lo = lax.iota(jnp.int32, 16)
    d_hi = d_lo + 16

    def col0_of(k):
        return pl.multiple_of((wid + k * NW) * CW, CW)

    def src_slice(slot):
        return src_v.at[pl.ds(pl.multiple_of(slot * DIM, DIM), DIM), :]

    def dst_slice(slot):
        return dst_v.at[pl.ds(pl.multiple_of(slot * (CW * HDIM), 8),
                              CW * HDIM)]

    def start_load(k, slot):
        pltpu.async_copy(tabt_hbm.at[:, pl.ds(col0_of(k), CW)],
                         src_slice(slot), lsem.at[slot])

    def wait_load(k, slot):
        pltpu.make_async_copy(tabt_hbm.at[:, pl.ds(col0_of(k), CW)],
                              src_slice(slot), lsem.at[slot]).wait()

    lane = lax.iota(jnp.int32, 16)
    sc_base = lane * SKEW

    def transpose_rows(src, d_base, dst_base, n16):
        G = 4

        def blk(r16, carry):
            c0 = r16 * 16
            prev = None
            for g0 in range(0, 16 + G, G):
                cur = None
                if g0 < 16:
                    cur = []
                    for d in range(g0, g0 + G):
                        a = src[d_base + d, pl.ds(c0, 16)]
                        b = src[d_base + d + 16, pl.ds(c0, 16)]
                        pw = plsc.bitcast(
                            plsc.pack(a, b,
                                      format=plsc.PackFormat.INTERLEAVED),
                            jnp.int32)
                        cur.append((d, pw))
                if prev is not None:
                    for d, pw in prev:
                        plsc.store_scatter(tmp_v, [sc_base + d], pw)
                prev = cur
            prevw = None
            for t0 in range(0, 16 + G, G):
                curw = None
                if t0 < 16:
                    curw = [(t, plsc.load_gather(tmp_v, [lane + t * SKEW]))
                            for t in range(t0, t0 + G)]
                if prevw is not None:
                    for t, w in prevw:
                        dst_v[pl.ds(dst_base + (c0 + t) * HDIM, 16)] = w
                prevw = curw
            return carry

        lax.fori_loop(0, n16, blk, 0)

    def start_out(k, slot):
        pltpu.async_copy(dst_slice(slot),
                         flat_hbm.at[pl.ds(col0_of(k) * HDIM, CW * HDIM)],
                         osem.at[slot])

    def wait_out(k, slot):
        pltpu.make_async_copy(dst_slice(slot),
                              flat_hbm.at[pl.ds(col0_of(k) * HDIM, CW * HDIM)],
                              osem.at[slot]).wait()

    start_load(0, 0)

    def body(k, carry):
        slot = k % 2
        wait_load(k, slot)

        @pl.when(k + 1 < K_W)
        def _():
            start_load(k + 1, (k + 1) % 2)

        @pl.when(k >= 2)
        def _():
            wait_out(k - 2, slot)

        transpose_rows(src_v, slot * DIM, slot * (CW * HDIM), CW // 16)
        start_out(k, slot)
        return carry

    lax.fori_loop(0, K_W, body, 0)
    wait_out(K_W - 2, (K_W - 2) % 2)
    wait_out(K_W - 1, (K_W - 1) % 2)

    @pl.when(wid == 16)
    def _():
        c0 = EXTRA_CID * CW
        pltpu.sync_copy(tabt_hbm.at[:, pl.ds(c0, CW)], src_slice(0))
        transpose_rows(src_v, 0, 0, CW // 16)
        pltpu.sync_copy(dst_slice(0),
                        flat_hbm.at[pl.ds(c0 * HDIM, CW * HDIM)])
        pltpu.sync_copy(tail_hbm, tail_v)
        transpose_rows(tail_v, 0, 0, 128 // 16)
        pltpu.sync_copy(dst_v.at[pl.ds(0, 128 * HDIM)],
                        flat_hbm.at[pl.ds(TAIL_SRC0 * HDIM, 128 * HDIM)])


_format = functools.partial(
    pl.kernel,
    mesh=plsc.VectorSubcoreMesh(core_axis_name="c", subcore_axis_name="s"),
    out_type=jax.ShapeDtypeStruct((VOCAB * HDIM,), jnp.int32),
    scratch_types=[
        pltpu.VMEM((2 * DIM, CW), jnp.float32),
        pltpu.VMEM((DIM, 128), jnp.float32),
        pltpu.VMEM((2 * CW * HDIM,), jnp.int32),
        pltpu.VMEM((16 * SKEW,), jnp.int32),
        pltpu.SemaphoreType.DMA((2,)),
        pltpu.SemaphoreType.DMA((2,)),
    ],
    compiler_params=pltpu.CompilerParams(use_tc_tiling_on_sc=True,
                                         needs_layout_passes=False),
)(_format_body)


def _pool_body(x_hbm, table_hbm, out_hbm, idx_v, rows_v, pooled_v, gsem):
    wid = lax.axis_index("s") * NC + lax.axis_index("c")
    ebase = wid * E_W

    def stage(c, slot):
        e0 = ebase + c * CHUNK_E
        ib = pl.multiple_of(slot * CHUNK_E, CHUNK_E)
        pltpu.sync_copy(x_hbm.at[pl.ds(e0, CHUNK_E), :],
                        idx_v.at[pl.ds(ib, CHUNK_E), :])
        for j in range(CHUNK_E):
            pltpu.async_copy(
                table_hbm.at[idx_v.at[ib + j]],
                rows_v.at[pl.ds(pl.multiple_of(slot * ROWS_C, 8) + j * L, L)],
                gsem.at[slot])

    def drain(slot):
        pltpu.make_async_copy(
            table_hbm.at[pl.ds(0, ROWS_C)],
            rows_v.at[pl.ds(pl.multiple_of(slot * ROWS_C, 8), ROWS_C)],
            gsem.at[slot]).wait()

    stage(0, 0)

    def chunk_body(c, carry):
        slot = c % 2
        e0 = ebase + c * CHUNK_E

        @pl.when(c + 1 < N_CHUNK)
        def _():
            stage(c + 1, (c + 1) % 2)

        drain(slot)
        rbase = slot * ROWS_C

        def elem_body(e, carry2):
            base = rbase + e * L
            acc0, acc1 = plsc.unpack(
                plsc.bitcast(rows_v[base, :], jnp.bfloat16),
                format=plsc.PackFormat.INTERLEAVED)
            for r in range(1, L):
                a, b = plsc.unpack(
                    plsc.bitcast(rows_v[base + r, :], jnp.bfloat16),
                    format=plsc.PackFormat.INTERLEAVED)
                acc0 = acc0 + a
                acc1 = acc1 + b
            pooled_v[e, pl.ds(0, 16)] = acc0 * (1.0 / L)
            pooled_v[e, pl.ds(16, 16)] = acc1 * (1.0 / L)
            return carry2

        lax.fori_loop(0, CHUNK_E, elem_body, 0)
        pltpu.sync_copy(pooled_v, out_hbm.at[pl.ds(e0, CHUNK_E)])
        return carry

    lax.fori_loop(0, N_CHUNK, chunk_body, 0)


_pool = functools.partial(
    pl.kernel,
    mesh=plsc.VectorSubcoreMesh(core_axis_name="c", subcore_axis_name="s"),
    out_type=jax.ShapeDtypeStruct((B, DIM), jnp.float32),
    scratch_types=[
        pltpu.VMEM((2 * CHUNK_E, L), jnp.int32),
        pltpu.VMEM((2 * ROWS_C, HDIM), jnp.int32),
        pltpu.VMEM((CHUNK_E, DIM), jnp.float32),
        pltpu.SemaphoreType.DMA((2,)),
    ],
    compiler_params=pltpu.CompilerParams(use_tc_tiling_on_sc=False,
                                         needs_layout_passes=False),
)(_pool_body)


def _tail_body(pooled_ref, w_ref, b_ref, gamma_ref, beta_ref, mean_ref,
               var_ref, out_ref):
    p = pooled_ref[...]
    h = jnp.dot(p, w_ref[...], preferred_element_type=jnp.float32) + b_ref[...]
    scale = gamma_ref[...] * lax.rsqrt(var_ref[...] + 1e-3)
    h = (h - mean_ref[...]) * scale + beta_ref[...]
    nrm = lax.rsqrt(jnp.maximum(jnp.sum(h * h, axis=1, keepdims=True), 1e-12))
    out_ref[...] = h * nrm


def _tail(pooled, w, b, gamma, beta, mean, var):
    blk = 2048
    vec = pl.BlockSpec((1, DIM), lambda i: (0, 0))
    return pl.pallas_call(
        _tail_body,
        grid=(B // blk,),
        in_specs=[
            pl.BlockSpec((blk, DIM), lambda i: (i, 0)),
            pl.BlockSpec((DIM, DIM), lambda i: (0, 0)),
            vec, vec, vec, vec, vec,
        ],
        out_specs=pl.BlockSpec((blk, DIM), lambda i: (i, 0)),
        out_shape=jax.ShapeDtypeStruct((B, DIM), jnp.float32),
    )(pooled, w, b, gamma, beta, mean, var)


def kernel(x, table, W, b, gamma, beta, moving_mean, moving_var):
    tabt = jnp.swapaxes(table, 0, 1)
    flat = _format(tabt, lax.slice(tabt, (0, TAIL_SRC0), (DIM, VOCAB)))
    pooled = _pool(x.astype(jnp.int32), flat.reshape(VOCAB, HDIM))
    r = lambda v: v.reshape(1, DIM)
    return _tail(pooled, W, r(b), r(gamma), r(beta), r(moving_mean),
                 r(moving_var))
